# baseline (speedup 1.0000x reference)
import functools

import jax
import jax.numpy as jnp
from jax import lax
from jax.experimental import pallas as pl
from jax.experimental.pallas import tpu as pltpu
from jax.experimental.pallas import tpu_sc as plsc

B, N, V, DIN, DOUT = 4096, 50, 1000000, 64, 64
_EW = 128

_TCB = 4096


def _tr_body(wt_ref, wp_ref):
    wp_ref[:, :DIN] = wt_ref[...].T


def _tc_transpose(wt):
    grid = ((V + _TCB - 1) // _TCB,)
    return pl.pallas_call(
        _tr_body,
        grid=grid,
        in_specs=[pl.BlockSpec((DIN, _TCB), lambda i: (0, i))],
        out_specs=pl.BlockSpec((_TCB, _EW), lambda i: (i, 0)),
        out_shape=jax.ShapeDtypeStruct((V, _EW), jnp.float32),
    )(wt)


_NC, _NS = 2, 16
_NW = _NC * _NS
_ROWS = B * N
_PER_W = _ROWS // _NW
_CHUNK = 128
_NCHUNK = _PER_W // _CHUNK


@functools.cache
def _make_sc_gather():
    mesh = plsc.VectorSubcoreMesh(core_axis_name="c", subcore_axis_name="s")

    @functools.partial(
        pl.kernel,
        mesh=mesh,
        compiler_params=pltpu.CompilerParams(
            use_tc_tiling_on_sc=True, needs_layout_passes=False
        ),
        out_type=jax.ShapeDtypeStruct((_ROWS, _EW), jnp.float32),
        scratch_types=[
            pltpu.VMEM((_PER_W,), jnp.int32),
            pltpu.VMEM((2, _CHUNK, _EW), jnp.float32),
            pltpu.SemaphoreType.DMA,
            pltpu.SemaphoreType.DMA,
        ],
    )
    def _sc_gather(x_hbm, wp_hbm, out_hbm, xv, rows_v, sem0, sem1):
        wid = lax.axis_index("s") * _NC + lax.axis_index("c")
        pltpu.sync_copy(x_hbm.at[wid], xv)
        base = wid * _PER_W
        sems = (sem0, sem1)

        def gdesc(j, slot):
            return pltpu.make_async_copy(
                wp_hbm.at[xv.at[pl.ds(j * _CHUNK, _CHUNK)]],
                rows_v.at[slot],
                sems[slot],
            )

        gdesc(0, 0).start()
        gdesc(1, 1).start()

        def body(g, carry):
            for slot in range(2):
                j = 2 * g + slot
                gdesc(j, slot).wait()
                pltpu.sync_copy(
                    rows_v.at[slot],
                    out_hbm.at[pl.ds(base + j * _CHUNK, _CHUNK)],
                )

                @pl.when(j + 2 < _NCHUNK)
                def _():
                    gdesc(j + 2, slot).start()

            return carry

        lax.fori_loop(0, _NCHUNK // 2, body, 0)

    return _sc_gather


_BB = 64
_TR = _BB * N


def _tc_body(e_ref, p_ref, sq_ref, w1_ref, b1_ref, w2_ref, b2_ref, o_ref):
    e = e_ref[:, :DIN]
    h = jnp.maximum(
        jnp.dot(e, w1_ref[...], preferred_element_type=jnp.float32)
        + b1_ref[...],
        0.0,
    )
    pooled = jnp.dot(p_ref[...], h, preferred_element_type=jnp.float32)
    y = jnp.dot(pooled, w2_ref[...], preferred_element_type=jnp.float32)
    o_ref[...] = y / sq_ref[...] + b2_ref[...]


def _tc_mlp(e2, pmat, sq2, W1, b1, W2, b2):
    nb = B // _BB
    return pl.pallas_call(
        _tc_body,
        grid=(nb,),
        in_specs=[
            pl.BlockSpec((_TR, _EW), lambda i: (i, 0)),
            pl.BlockSpec((_BB, _TR), lambda i: (0, 0)),
            pl.BlockSpec((_BB, 1), lambda i: (i, 0)),
            pl.BlockSpec((DIN, DOUT), lambda i: (0, 0)),
            pl.BlockSpec((1, DOUT), lambda i: (0, 0)),
            pl.BlockSpec((DOUT, DOUT), lambda i: (0, 0)),
            pl.BlockSpec((1, DOUT), lambda i: (0, 0)),
        ],
        out_specs=pl.BlockSpec((_BB, DOUT), lambda i: (i, 0)),
        out_shape=jax.ShapeDtypeStruct((B, DOUT), jnp.float32),
    )(e2, pmat, sq2, W1, b1, W2, b2)


def kernel(x, sq_lengths, weight, W1, b1, W2, b2):
    wp = _tc_transpose(weight.T)
    x2 = x.reshape(_NW, _PER_W)
    e2 = _make_sc_gather()(x2, wp)
    pmat = (
        jnp.arange(_BB, dtype=jnp.int32)[:, None]
        == (jnp.arange(_TR, dtype=jnp.int32)[None, :] // N)
    ).astype(jnp.float32)
    return _tc_mlp(
        e2,
        pmat,
        sq_lengths.reshape(B, 1),
        W1,
        b1.reshape(1, DOUT),
        W2,
        b2.reshape(1, DOUT),
    )


# speedup vs baseline: 1.0412x; 1.0412x over previous
import functools

import jax
import jax.numpy as jnp
from jax import lax
from jax.experimental import pallas as pl
from jax.experimental.pallas import tpu as pltpu
from jax.experimental.pallas import tpu_sc as plsc

B, N, V, DIN, DOUT = 4096, 50, 1000000, 64, 64
_EW = 128

_TCB = 4096


def _tr_body(wt_ref, wp_ref):
    wp_ref[:, :DIN] = wt_ref[...].T


def _tc_transpose(wt):
    grid = ((V + _TCB - 1) // _TCB,)
    return pl.pallas_call(
        _tr_body,
        grid=grid,
        in_specs=[pl.BlockSpec((DIN, _TCB), lambda i: (0, i))],
        out_specs=pl.BlockSpec((_TCB, _EW), lambda i: (i, 0)),
        out_shape=jax.ShapeDtypeStruct((V, _EW), jnp.float32),
    )(wt)


_NC, _NS = 2, 16
_NW = _NC * _NS
_ROWS = B * N
_PER_W = _ROWS // _NW
_CHUNK = 128
_NCHUNK = _PER_W // _CHUNK


@functools.cache
def _make_sc_gather():
    mesh = plsc.VectorSubcoreMesh(core_axis_name="c", subcore_axis_name="s")

    @functools.partial(
        pl.kernel,
        mesh=mesh,
        compiler_params=pltpu.CompilerParams(
            use_tc_tiling_on_sc=True, needs_layout_passes=False
        ),
        out_type=jax.ShapeDtypeStruct((_ROWS, _EW), jnp.float32),
        scratch_types=[
            pltpu.VMEM((_PER_W,), jnp.int32),
            pltpu.VMEM((2, _CHUNK, _EW), jnp.float32),
            pltpu.SemaphoreType.DMA,
            pltpu.SemaphoreType.DMA,
        ],
    )
    def _sc_gather(x_hbm, wp_hbm, out_hbm, xv, rows_v, sem0, sem1):
        wid = lax.axis_index("s") * _NC + lax.axis_index("c")
        pltpu.sync_copy(x_hbm.at[wid], xv)
        base = wid * _PER_W
        sems = (sem0, sem1)

        def gdesc(j, slot):
            return pltpu.make_async_copy(
                wp_hbm.at[xv.at[pl.ds(j * _CHUNK, _CHUNK)]],
                rows_v.at[slot],
                sems[slot],
            )

        gdesc(0, 0).start()
        gdesc(1, 1).start()

        def body(g, carry):
            for slot in range(2):
                j = 2 * g + slot
                gdesc(j, slot).wait()
                pltpu.sync_copy(
                    rows_v.at[slot],
                    out_hbm.at[pl.ds(base + j * _CHUNK, _CHUNK)],
                )

                @pl.when(j + 2 < _NCHUNK)
                def _():
                    gdesc(j + 2, slot).start()

            return carry

        lax.fori_loop(0, _NCHUNK // 2, body, 0)

    return _sc_gather


_BB = 128
_TR = _BB * N


def _tc_body(e_ref, p_ref, sq_ref, w1_ref, b1_ref, w2_ref, b2_ref, o_ref):
    e = e_ref[:, :DIN]
    h = jnp.maximum(
        jnp.dot(e, w1_ref[...], preferred_element_type=jnp.float32)
        + b1_ref[...],
        0.0,
    )
    pooled = jnp.dot(p_ref[...], h, preferred_element_type=jnp.float32)
    y = jnp.dot(pooled, w2_ref[...], preferred_element_type=jnp.float32)
    o_ref[...] = y / sq_ref[...] + b2_ref[...]


def _tc_mlp(e2, pmat, sq2, W1, b1, W2, b2):
    nb = B // _BB
    return pl.pallas_call(
        _tc_body,
        grid=(nb,),
        in_specs=[
            pl.BlockSpec((_TR, _EW), lambda i: (i, 0)),
            pl.BlockSpec((_BB, _TR), lambda i: (0, 0)),
            pl.BlockSpec((_BB, 1), lambda i: (i, 0)),
            pl.BlockSpec((DIN, DOUT), lambda i: (0, 0)),
            pl.BlockSpec((1, DOUT), lambda i: (0, 0)),
            pl.BlockSpec((DOUT, DOUT), lambda i: (0, 0)),
            pl.BlockSpec((1, DOUT), lambda i: (0, 0)),
        ],
        out_specs=pl.BlockSpec((_BB, DOUT), lambda i: (i, 0)),
        out_shape=jax.ShapeDtypeStruct((B, DOUT), jnp.float32),
    )(e2, pmat, sq2, W1, b1, W2, b2)


def kernel(x, sq_lengths, weight, W1, b1, W2, b2):
    wp = _tc_transpose(weight.T)
    x2 = x.reshape(_NW, _PER_W)
    e2 = _make_sc_gather()(x2, wp)
    pmat = (
        jnp.arange(_BB, dtype=jnp.int32)[:, None]
        == (jnp.arange(_TR, dtype=jnp.int32)[None, :] // N)
    ).astype(jnp.float32)
    return _tc_mlp(
        e2,
        pmat,
        sq_lengths.reshape(B, 1),
        W1,
        b1.reshape(1, DOUT),
        W2,
        b2.reshape(1, DOUT),
    )


# speedup vs baseline: 1.1485x; 1.1030x over previous
import functools

import jax
import jax.numpy as jnp
from jax import lax
from jax.experimental import pallas as pl
from jax.experimental.pallas import tpu as pltpu
from jax.experimental.pallas import tpu_sc as plsc

B, N, V, DIN, DOUT = 4096, 50, 1000000, 64, 64
_EW = 128

_TCB = 6144


def _tr_body(wt_ref, wp_ref):
    wp_ref[:, :DIN] = wt_ref[...].T


def _tc_transpose(wt):
    grid = ((V + _TCB - 1) // _TCB,)
    return pl.pallas_call(
        _tr_body,
        grid=grid,
        in_specs=[pl.BlockSpec((DIN, _TCB), lambda i: (0, i))],
        out_specs=pl.BlockSpec((_TCB, _EW), lambda i: (i, 0)),
        out_shape=jax.ShapeDtypeStruct((V, _EW), jnp.float32),
    )(wt)


_NC, _NS = 2, 16
_NW = _NC * _NS
_ROWS = B * N
_PER_W = _ROWS // _NW
_CHUNK = 128
_NCHUNK = _PER_W // _CHUNK


@functools.cache
def _make_sc_gather():
    mesh = plsc.VectorSubcoreMesh(core_axis_name="c", subcore_axis_name="s")

    @functools.partial(
        pl.kernel,
        mesh=mesh,
        compiler_params=pltpu.CompilerParams(
            use_tc_tiling_on_sc=True, needs_layout_passes=False
        ),
        out_type=jax.ShapeDtypeStruct((_ROWS, _EW), jnp.float32),
        scratch_types=[
            pltpu.VMEM((_PER_W,), jnp.int32),
            pltpu.VMEM((2, _CHUNK, _EW), jnp.float32),
            pltpu.SemaphoreType.DMA,
            pltpu.SemaphoreType.DMA,
        ],
    )
    def _sc_gather(x_hbm, wp_hbm, out_hbm, xv, rows_v, sem0, sem1):
        wid = lax.axis_index("s") * _NC + lax.axis_index("c")
        pltpu.sync_copy(x_hbm.at[wid], xv)
        base = wid * _PER_W
        sems = (sem0, sem1)

        def gdesc(j, slot):
            return pltpu.make_async_copy(
                wp_hbm.at[xv.at[pl.ds(j * _CHUNK, _CHUNK)]],
                rows_v.at[slot],
                sems[slot],
            )

        gdesc(0, 0).start()
        gdesc(1, 1).start()

        def body(g, carry):
            for slot in range(2):
                j = 2 * g + slot
                gdesc(j, slot).wait()
                pltpu.sync_copy(
                    rows_v.at[slot],
                    out_hbm.at[pl.ds(base + j * _CHUNK, _CHUNK)],
                )

                @pl.when(j + 2 < _NCHUNK)
                def _():
                    gdesc(j + 2, slot).start()

            return carry

        lax.fori_loop(0, _NCHUNK // 2, body, 0)

    return _sc_gather


_BB = 128
_TR = _BB * N


def _tc_body(e_ref, p_ref, sq_ref, w1_ref, b1_ref, w2_ref, b2_ref, o_ref):
    e = e_ref[:, :DIN]
    h = jnp.maximum(
        jnp.dot(e, w1_ref[...], preferred_element_type=jnp.float32)
        + b1_ref[...],
        0.0,
    )
    pooled = jnp.dot(p_ref[...], h, preferred_element_type=jnp.float32)
    y = jnp.dot(pooled, w2_ref[...], preferred_element_type=jnp.float32)
    o_ref[...] = y / sq_ref[...] + b2_ref[...]


def _tc_mlp(e2, pmat, sq2, W1, b1, W2, b2):
    nb = B // _BB
    return pl.pallas_call(
        _tc_body,
        grid=(nb,),
        in_specs=[
            pl.BlockSpec((_TR, _EW), lambda i: (i, 0)),
            pl.BlockSpec((_BB, _TR), lambda i: (0, 0)),
            pl.BlockSpec((_BB, 1), lambda i: (i, 0)),
            pl.BlockSpec((DIN, DOUT), lambda i: (0, 0)),
            pl.BlockSpec((1, DOUT), lambda i: (0, 0)),
            pl.BlockSpec((DOUT, DOUT), lambda i: (0, 0)),
            pl.BlockSpec((1, DOUT), lambda i: (0, 0)),
        ],
        out_specs=pl.BlockSpec((_BB, DOUT), lambda i: (i, 0)),
        out_shape=jax.ShapeDtypeStruct((B, DOUT), jnp.float32),
    )(e2, pmat, sq2, W1, b1, W2, b2)


def kernel(x, sq_lengths, weight, W1, b1, W2, b2):
    wp = _tc_transpose(weight.T)
    x2 = x.reshape(_NW, _PER_W)
    e2 = _make_sc_gather()(x2, wp)
    pmat = (
        jnp.arange(_BB, dtype=jnp.int32)[:, None]
        == (jnp.arange(_TR, dtype=jnp.int32)[None, :] // N)
    ).astype(jnp.float32)
    return _tc_mlp(
        e2,
        pmat,
        sq_lengths.reshape(B, 1),
        W1,
        b1.reshape(1, DOUT),
        W2,
        b2.reshape(1, DOUT),
    )
